# baseline (speedup 1.0000x reference)
import functools

import jax
import jax.numpy as jnp
from jax import lax
from jax.experimental import pallas as pl
from jax.experimental.pallas import tpu as pltpu, tpu_sc as plsc

B = 16384
DIM = 64
CH = 64
NBUF = 2


@functools.lru_cache(maxsize=None)
def _build(num_cores, num_subcores):
    NW = num_cores * num_subcores
    b_per_w = B // NW
    NCH = b_per_w // CH
    G = CH // 16
    UG = b_per_w // 16
    WCH = 128

    mesh = plsc.VectorSubcoreMesh(core_axis_name="c", subcore_axis_name="s")
    out_sds = jax.ShapeDtypeStruct((B, DIM), jnp.float32)

    @functools.partial(
        pl.kernel,
        mesh=mesh,
        out_type=(out_sds, out_sds, out_sds),
        scratch_types=[
            pltpu.VMEM((b_per_w,), jnp.int32),
            pltpu.VMEM((b_per_w,), jnp.int32),
            pltpu.VMEM((b_per_w,), jnp.int32),
            pltpu.VMEM((b_per_w,), jnp.int32),
            pltpu.VMEM((b_per_w,), jnp.int32),
            pltpu.VMEM((b_per_w, DIM), jnp.float32),
            [pltpu.VMEM((CH, 2 * DIM), jnp.float32) for _ in range(NBUF)],
            [pltpu.VMEM((CH, DIM), jnp.float32) for _ in range(NBUF)],
            pltpu.SemaphoreType.DMA,
            pltpu.SemaphoreType.DMA,
            [pltpu.SemaphoreType.DMA for _ in range(NBUF)],
            [pltpu.SemaphoreType.DMA for _ in range(NBUF)],
        ],
        compiler_params=pltpu.CompilerParams(needs_layout_passes=False),
    )
    def sc_gather3(u_i, p_p, p_h, n_p, n_h, utab, itab,
                   out_u, out_p, out_n,
                   uidx, ppv, phv, npv, nhv, ubuf, pairs, rows,
                   ugsem, uwsem, gsems, wsems):
        wid = lax.axis_index("s") * num_cores + lax.axis_index("c")
        base = wid * b_per_w

        pltpu.sync_copy(u_i.at[wid], uidx)
        pltpu.sync_copy(p_p.at[wid], ppv)
        pltpu.sync_copy(p_h.at[wid], phv)
        pltpu.sync_copy(n_p.at[wid], npv)
        pltpu.sync_copy(n_h.at[wid], nhv)

        def u_issue(g, carry):
            v = uidx[pl.ds(g * 16, 16)]
            for l in range(16):
                pltpu.async_copy(utab.at[pl.ds(v[l], 1)],
                                 ubuf.at[pl.ds(g * 16 + l, 1)], ugsem)
            return carry

        lax.fori_loop(0, UG, u_issue, 0)

        sched = []
        for pv, hv, out in ((ppv, phv, out_p), (npv, nhv, out_n)):
            for c in range(NCH):
                sched.append((pv, hv, out, c * CH))
        total = len(sched)

        def fire(slot):
            pv, _, _, ofs = sched[slot]
            pltpu.async_copy(itab.at[pv.at[pl.ds(ofs, CH)]],
                             pairs[slot % NBUF], gsems[slot % NBUF])

        def drain_gather(slot):
            pv = sched[slot][0]
            pltpu.make_async_copy(itab.at[pv.at[pl.ds(0, CH)]],
                                  pairs[slot % NBUF], gsems[slot % NBUF]).wait()

        def extract(slot):
            _, hv, _, ofs = sched[slot]
            pbuf = pairs[slot % NBUF]
            rbuf = rows[slot % NBUF]

            def group(g, carry):
                jrow = lax.iota(jnp.int32, 16) + g * 16
                hvec = hv[pl.ds(ofs + g * 16, 16)]
                for col in range(DIM):
                    x = plsc.load_gather(pbuf, [jrow, hvec + col])
                    plsc.store_scatter(rbuf, [jrow, jnp.full((16,), col, jnp.int32)], x)
                return carry

            lax.fori_loop(0, G, group, 0)

        def start_writeback(slot):
            _, _, out, ofs = sched[slot]
            pltpu.async_copy(rows[slot % NBUF], out.at[pl.ds(base + ofs, CH)],
                             wsems[slot % NBUF])

        def drain_writeback(slot):
            _, _, out, ofs = sched[slot]
            pltpu.make_async_copy(rows[slot % NBUF], out.at[pl.ds(base + ofs, CH)],
                                  wsems[slot % NBUF]).wait()

        for s in range(min(NBUF - 1, total)):
            fire(s)
        for s in range(total):
            drain_gather(s)
            if s >= NBUF:
                drain_writeback(s - NBUF)
            extract(s)
            start_writeback(s)
            nxt = s + NBUF - 1
            if nxt < total:
                fire(nxt)
        for s in range(max(total - NBUF, 0), total):
            drain_writeback(s)

        def u_drain(j, carry):
            pltpu.make_async_copy(utab.at[pl.ds(0, 1)],
                                  ubuf.at[pl.ds(0, 1)], ugsem).wait()
            return carry

        lax.fori_loop(0, b_per_w, u_drain, 0)
        for c in range(b_per_w // WCH):
            pltpu.async_copy(ubuf.at[pl.ds(c * WCH, WCH)],
                             out_u.at[pl.ds(base + c * WCH, WCH)], uwsem)
        for c in range(b_per_w // WCH):
            pltpu.make_async_copy(ubuf.at[pl.ds(c * WCH, WCH)],
                                  out_u.at[pl.ds(base + c * WCH, WCH)],
                                  uwsem).wait()

    return sc_gather3, NW, b_per_w


def kernel(user, pos, neg, user_table, item_table):
    info = plsc.get_sparse_core_info()
    fn, nw, bw = _build(info.num_cores, info.num_subcores)

    def split(idx):
        idx = idx.astype(jnp.int32)
        return ((idx >> 1).reshape(nw, bw),
                ((idx & 1) * DIM).reshape(nw, bw))

    u = user.astype(jnp.int32).reshape(nw, bw)
    p_p, p_h = split(pos)
    n_p, n_h = split(neg)
    it2 = item_table.reshape(item_table.shape[0] // 2, 2 * DIM)
    return fn(u, p_p, p_h, n_p, n_h, user_table, it2)
